# Initial kernel scaffold; baseline (speedup 1.0000x reference)
import functools

import jax
import jax.numpy as jnp
from jax import lax
from jax.experimental import pallas as pl
from jax.experimental.pallas import tpu as pltpu
from jax.experimental.pallas import tpu_sc as plsc

BATCH = 4096
SEQ = 50
DIM_WORD = 128
DIM_FEAT = 64
DIM_OUT = DIM_WORD + DIM_FEAT

N = BATCH * SEQ
NUM_CORES = 2
NUM_SUBCORES = 16
NW = NUM_CORES * NUM_SUBCORES
PER_W = N // NW
CHUNK = 128
NCH = PER_W // CHUNK

_mesh = plsc.VectorSubcoreMesh(core_axis_name="c", subcore_axis_name="s")


@functools.partial(
    pl.kernel,
    mesh=_mesh,
    out_type=jax.ShapeDtypeStruct((N, DIM_OUT), jnp.float32),
    scratch_types=[
        pltpu.VMEM((NCH, CHUNK), jnp.int32),
        pltpu.VMEM((NCH, CHUNK), jnp.int32),
        pltpu.VMEM((CHUNK, DIM_WORD), jnp.float32),
        pltpu.VMEM((CHUNK, DIM_FEAT), jnp.float32),
        pltpu.SemaphoreType.DMA,
    ],
)
def _emb_stack(word_hbm, feat_hbm, w_word_hbm, w_feat_hbm, out_hbm,
               widx, fidx, wrows, frows, sem):
    wid = lax.axis_index("s") * NUM_CORES + lax.axis_index("c")
    base = wid * PER_W

    pltpu.sync_copy(word_hbm.at[wid], widx)
    pltpu.sync_copy(feat_hbm.at[wid], fidx)

    def chunk_body(j, carry):
        row0 = base + j * CHUNK
        cw = pltpu.async_copy(w_word_hbm.at[widx.at[j]], wrows, sem)
        cf = pltpu.async_copy(w_feat_hbm.at[fidx.at[j]], frows, sem)
        cw.wait()
        cf.wait()
        pltpu.sync_copy(wrows, out_hbm.at[pl.ds(row0, CHUNK), pl.ds(0, DIM_WORD)])
        pltpu.sync_copy(frows, out_hbm.at[pl.ds(row0, CHUNK), pl.ds(DIM_WORD, DIM_FEAT)])
        return carry

    lax.fori_loop(0, NCH, chunk_body, 0)


def kernel(word, feat, W_word, W_feat):
    word_i = word.astype(jnp.int32).reshape(NW, NCH, CHUNK)
    feat_i = feat.astype(jnp.int32).reshape(NW, NCH, CHUNK)
    out = _emb_stack(word_i, feat_i, W_word, W_feat)
    return out.reshape(BATCH, SEQ, DIM_OUT)


# speedup vs baseline: 3.8470x; 3.8470x over previous
import functools

import jax
import jax.numpy as jnp
from jax import lax
from jax.experimental import pallas as pl
from jax.experimental.pallas import tpu as pltpu
from jax.experimental.pallas import tpu_sc as plsc

BATCH = 4096
SEQ = 50
DIM_WORD = 128
DIM_FEAT = 64
DIM_OUT = DIM_WORD + DIM_FEAT

N = BATCH * SEQ
NUM_CORES = 2
NUM_SUBCORES = 16
NW = NUM_CORES * NUM_SUBCORES
PER_W = N // NW
CHUNK = 128
NCH = PER_W // CHUNK

_mesh = plsc.VectorSubcoreMesh(core_axis_name="c", subcore_axis_name="s")


@functools.partial(
    pl.kernel,
    mesh=_mesh,
    compiler_params=pltpu.CompilerParams(use_tc_tiling_on_sc=False),
    out_type=jax.ShapeDtypeStruct((N, DIM_OUT), jnp.float32),
    scratch_types=[
        pltpu.VMEM((NCH, CHUNK), jnp.int32),
        pltpu.VMEM((NCH, CHUNK), jnp.int32),
        pltpu.VMEM((CHUNK, DIM_WORD), jnp.float32),
        pltpu.VMEM((CHUNK, DIM_FEAT), jnp.float32),
        pltpu.SemaphoreType.DMA,
    ],
)
def _emb_stack(word_hbm, feat_hbm, w_word_hbm, w_feat_hbm, out_hbm,
               widx, fidx, wrows, frows, sem):
    wid = lax.axis_index("s") * NUM_CORES + lax.axis_index("c")
    base = wid * PER_W

    pltpu.sync_copy(word_hbm.at[wid], widx)
    pltpu.sync_copy(feat_hbm.at[wid], fidx)

    def chunk_body(j, carry):
        row0 = base + j * CHUNK
        cw = pltpu.async_copy(w_word_hbm.at[widx.at[j]], wrows, sem)
        cf = pltpu.async_copy(w_feat_hbm.at[fidx.at[j]], frows, sem)
        cw.wait()
        cf.wait()
        pltpu.sync_copy(wrows, out_hbm.at[pl.ds(row0, CHUNK), pl.ds(0, DIM_WORD)])
        pltpu.sync_copy(frows, out_hbm.at[pl.ds(row0, CHUNK), pl.ds(DIM_WORD, DIM_FEAT)])
        return carry

    lax.fori_loop(0, NCH, chunk_body, 0)


def kernel(word, feat, W_word, W_feat):
    word_i = word.astype(jnp.int32).reshape(NW, NCH, CHUNK)
    feat_i = feat.astype(jnp.int32).reshape(NW, NCH, CHUNK)
    out = _emb_stack(word_i, feat_i, W_word, W_feat)
    return out.reshape(BATCH, SEQ, DIM_OUT)


# speedup vs baseline: 4.0060x; 1.0413x over previous
import functools

import jax
import jax.numpy as jnp
from jax import lax
from jax.experimental import pallas as pl
from jax.experimental.pallas import tpu as pltpu
from jax.experimental.pallas import tpu_sc as plsc

BATCH = 4096
SEQ = 50
DIM_WORD = 128
DIM_FEAT = 64
DIM_OUT = DIM_WORD + DIM_FEAT

N = BATCH * SEQ
NUM_CORES = 2
NUM_SUBCORES = 16
NW = NUM_CORES * NUM_SUBCORES
PER_W = N // NW
CHUNK = 128
NCH = PER_W // CHUNK

_mesh = plsc.VectorSubcoreMesh(core_axis_name="c", subcore_axis_name="s")


NBUF = 4


@functools.partial(
    pl.kernel,
    mesh=_mesh,
    compiler_params=pltpu.CompilerParams(use_tc_tiling_on_sc=False),
    out_type=jax.ShapeDtypeStruct((N, DIM_OUT), jnp.float32),
    scratch_types=[
        pltpu.VMEM((NCH, CHUNK), jnp.int32),
        pltpu.VMEM((NCH, CHUNK), jnp.int32),
        pltpu.VMEM((NBUF, CHUNK, DIM_WORD), jnp.float32),
        pltpu.VMEM((NBUF, CHUNK, DIM_FEAT), jnp.float32),
        pltpu.SemaphoreType.DMA((NBUF,)),
        pltpu.SemaphoreType.DMA((NBUF,)),
    ],
)
def _emb_stack(word_hbm, feat_hbm, w_word_hbm, w_feat_hbm, out_hbm,
               widx, fidx, wrows, frows, semw, semf):
    wid = lax.axis_index("s") * NUM_CORES + lax.axis_index("c")
    base = wid * PER_W

    pltpu.sync_copy(word_hbm.at[wid], widx)
    pltpu.sync_copy(feat_hbm.at[wid], fidx)

    def issue(j, b):
        pltpu.async_copy(w_word_hbm.at[widx.at[j]], wrows.at[b], semw.at[b])
        pltpu.async_copy(w_feat_hbm.at[fidx.at[j]], frows.at[b], semf.at[b])

    def drain_and_write(j, b):
        pltpu.make_async_copy(w_word_hbm.at[widx.at[j]], wrows.at[b], semw.at[b]).wait()
        pltpu.make_async_copy(w_feat_hbm.at[fidx.at[j]], frows.at[b], semf.at[b]).wait()
        row0 = base + j * CHUNK
        pltpu.sync_copy(wrows.at[b], out_hbm.at[pl.ds(row0, CHUNK), pl.ds(0, DIM_WORD)])
        pltpu.sync_copy(frows.at[b], out_hbm.at[pl.ds(row0, CHUNK), pl.ds(DIM_WORD, DIM_FEAT)])

    for b in range(NBUF):
        issue(b, b)

    def group_body(g, carry):
        j0 = g * NBUF
        for b in range(NBUF):
            j = j0 + b
            drain_and_write(j, b)

            @pl.when(j + NBUF < NCH)
            def _():
                issue(j + NBUF, b)
        return carry

    lax.fori_loop(0, NCH // NBUF, group_body, 0)

    for t in range(NCH - (NCH // NBUF) * NBUF):
        drain_and_write((NCH // NBUF) * NBUF + t, t)


def kernel(word, feat, W_word, W_feat):
    word_i = word.astype(jnp.int32).reshape(NW, NCH, CHUNK)
    feat_i = feat.astype(jnp.int32).reshape(NW, NCH, CHUNK)
    out = _emb_stack(word_i, feat_i, W_word, W_feat)
    return out.reshape(BATCH, SEQ, DIM_OUT)


# speedup vs baseline: 5.1809x; 1.2933x over previous
import functools

import jax
import jax.numpy as jnp
from jax import lax
from jax.experimental import pallas as pl
from jax.experimental.pallas import tpu as pltpu
from jax.experimental.pallas import tpu_sc as plsc

BATCH = 4096
SEQ = 50
DIM_WORD = 128
DIM_FEAT = 64
DIM_OUT = DIM_WORD + DIM_FEAT

NUM_CORES = 2
NUM_SUBCORES = 16
NW = NUM_CORES * NUM_SUBCORES
BPW = BATCH // NW
NBUF = 4

_mesh = plsc.VectorSubcoreMesh(core_axis_name="c", subcore_axis_name="s")


@functools.partial(
    pl.kernel,
    mesh=_mesh,
    compiler_params=pltpu.CompilerParams(use_tc_tiling_on_sc=True),
    out_type=jax.ShapeDtypeStruct((BATCH, SEQ, DIM_OUT), jnp.float32),
    scratch_types=[
        pltpu.VMEM((BPW, SEQ), jnp.int32),
        pltpu.VMEM((BPW, SEQ), jnp.int32),
        pltpu.VMEM((NBUF, SEQ, DIM_OUT), jnp.float32),
        pltpu.VMEM((NBUF, SEQ, DIM_WORD), jnp.float32),
        pltpu.SemaphoreType.DMA((NBUF,)),
        pltpu.SemaphoreType.DMA((NBUF,)),
    ],
)
def _emb_stack(word_hbm, feat_hbm, w_word_hbm, w_featp_hbm, out_hbm,
               widx, fidx, comb, frows, semw, semf):
    wid = lax.axis_index("s") * NUM_CORES + lax.axis_index("c")
    b0 = wid * BPW

    pltpu.sync_copy(word_hbm.at[wid], widx)
    pltpu.sync_copy(feat_hbm.at[wid], fidx)

    def issue(j, b):
        pltpu.async_copy(w_word_hbm.at[widx.at[j]],
                         comb.at[b, :, pl.ds(0, DIM_WORD)], semw.at[b])
        pltpu.async_copy(w_featp_hbm.at[fidx.at[j]], frows.at[b], semf.at[b])

    def drain_and_write(j, b):
        pltpu.make_async_copy(w_word_hbm.at[widx.at[j]],
                              comb.at[b, :, pl.ds(0, DIM_WORD)], semw.at[b]).wait()
        pltpu.make_async_copy(w_featp_hbm.at[fidx.at[j]], frows.at[b], semf.at[b]).wait()
        for s in range(SEQ):
            for k in range(DIM_FEAT // 16):
                comb[b, s, pl.ds(DIM_WORD + k * 16, 16)] = frows[b, s, pl.ds(k * 16, 16)]
        bi = b0 + j
        pltpu.sync_copy(comb.at[b], out_hbm.at[bi])

    for b in range(NBUF):
        issue(b, b)

    def group_body(g, carry):
        j0 = g * NBUF
        for b in range(NBUF):
            j = j0 + b
            drain_and_write(j, b)

            @pl.when(j + NBUF < BPW)
            def _():
                issue(j + NBUF, b)
        return carry

    lax.fori_loop(0, BPW // NBUF, group_body, 0)


def kernel(word, feat, W_word, W_feat):
    word3 = word.astype(jnp.int32).reshape(NW, BPW, SEQ)
    feat3 = feat.astype(jnp.int32).reshape(NW, BPW, SEQ)
    w_featp = jnp.pad(W_feat, ((0, 0), (0, DIM_WORD - DIM_FEAT)))
    return _emb_stack(word3, feat3, W_word, w_featp)


# speedup vs baseline: 11.3800x; 2.1966x over previous
import functools

import jax
import jax.numpy as jnp
from jax import lax
from jax.experimental import pallas as pl
from jax.experimental.pallas import tpu as pltpu
from jax.experimental.pallas import tpu_sc as plsc

BATCH = 4096
SEQ = 50
DIM_WORD = 128
DIM_FEAT = 64
DIM_OUT = DIM_WORD + DIM_FEAT

NUM_CORES = 2
NUM_SUBCORES = 16
NW = NUM_CORES * NUM_SUBCORES
BPW = BATCH // NW
NBUF = 3
LANES = 16
KW = DIM_WORD // LANES
KF = DIM_FEAT // LANES

_mesh = plsc.VectorSubcoreMesh(core_axis_name="c", subcore_axis_name="s")


@functools.partial(
    pl.kernel,
    mesh=_mesh,
    compiler_params=pltpu.CompilerParams(use_tc_tiling_on_sc=True,
                                         needs_layout_passes=False),
    out_type=jax.ShapeDtypeStruct((SEQ, DIM_OUT, BATCH), jnp.float32),
    scratch_types=[
        pltpu.VMEM((SEQ, BPW), jnp.int32),
        pltpu.VMEM((SEQ, BPW), jnp.int32),
        pltpu.VMEM((NBUF, BPW, DIM_WORD), jnp.float32),
        pltpu.VMEM((NBUF, BPW, DIM_WORD), jnp.float32),
        pltpu.VMEM((DIM_WORD * BPW,), jnp.float32),
        pltpu.SemaphoreType.DMA((NBUF,)),
        pltpu.SemaphoreType.DMA((NBUF,)),
        pltpu.SemaphoreType.DMA((NBUF,)),
    ],
)
def _emb_stack(word_hbm, feat_hbm, w_word_hbm, w_featp_hbm, out_hbm,
               widx, fidx, wbuf, fbuf, sw, semw, semf, sema):
    wid = lax.axis_index("s") * NUM_CORES + lax.axis_index("c")
    b0 = wid * BPW

    pltpu.sync_copy(word_hbm.at[wid], widx)
    pltpu.sync_copy(feat_hbm.at[wid], fidx)

    lane16 = lax.iota(jnp.int32, LANES)
    rowflat = [(lane16 + k * LANES) * BPW for k in range(KW)]

    def issue_gather(s, p):
        pltpu.async_copy(w_word_hbm.at[widx.at[s]], wbuf.at[p], semw.at[p])
        pltpu.async_copy(w_featp_hbm.at[fidx.at[s]], fbuf.at[p], semf.at[p])

    def wait_gather(s, p):
        pltpu.make_async_copy(w_word_hbm.at[widx.at[s]], wbuf.at[p], semw.at[p]).wait()
        pltpu.make_async_copy(w_featp_hbm.at[fidx.at[s]], fbuf.at[p], semf.at[p]).wait()

    def word_write(s, p):
        return (wbuf.at[p], out_hbm.at[s, pl.ds(0, DIM_WORD), pl.ds(b0, BPW)])

    def feat_write(s, p):
        return (fbuf.at[p, pl.ds(0, DIM_FEAT)],
                out_hbm.at[s, pl.ds(DIM_WORD, DIM_FEAT), pl.ds(b0, BPW)])

    def issue_write(s, p):
        pltpu.async_copy(*word_write(s, p), sema.at[p])
        pltpu.async_copy(*feat_write(s, p), sema.at[p])

    def wait_write(s, p):
        pltpu.make_async_copy(*word_write(s, p), sema.at[p]).wait()
        pltpu.make_async_copy(*feat_write(s, p), sema.at[p]).wait()

    def scatter_pass(buf, p, nk):
        @plsc.parallel_loop(0, BPW, step=1, unroll=2)
        def _(b):
            colx = jnp.bitwise_xor(lax.broadcast(b, (LANES,)), lane16)
            for k in range(nk):
                v = buf[p, b, pl.ds(k * LANES, LANES)]
                plsc.store_scatter(sw, [rowflat[k] + colx], v)

    def unswizzle_pass(buf, p, nd):
        @plsc.parallel_loop(0, nd, step=1, unroll=2)
        def _(d):
            dl = jnp.bitwise_and(d, LANES - 1)
            colx = jnp.bitwise_xor(lax.broadcast(dl, (LANES,)), lane16)
            row = lax.broadcast(d, (LANES,))
            for j in range(BPW // LANES):
                v = sw[pl.ds(d * BPW + j * LANES, LANES)]
                plsc.store_scatter(buf.at[p], [row, colx + j * LANES], v)

    def transpose_slab(p):
        scatter_pass(wbuf, p, KW)
        unswizzle_pass(wbuf, p, DIM_WORD)
        scatter_pass(fbuf, p, KF)
        unswizzle_pass(fbuf, p, DIM_FEAT)

    for p in range(2):
        issue_gather(p, p)

    def do_slab(s, p):
        wait_gather(s, p)
        transpose_slab(p)
        issue_write(s, p)
        t = s + 2
        q = (p + 2) % NBUF

        @pl.when(t < SEQ)
        def _():
            @pl.when(t >= NBUF)
            def _():
                wait_write(t - NBUF, q)
            issue_gather(t, q)

    def group_body(g, carry):
        for p in range(NBUF):
            do_slab(g * NBUF + p, p)
        return carry

    lax.fori_loop(0, SEQ // NBUF, group_body, 0)
    for p in range(SEQ - (SEQ // NBUF) * NBUF):
        do_slab((SEQ // NBUF) * NBUF + p, p)

    for s in range(SEQ - NBUF, SEQ):
        wait_write(s, s % NBUF)


def kernel(word, feat, W_word, W_feat):
    wordT = word.astype(jnp.int32).reshape(NW, BPW, SEQ).transpose(0, 2, 1)
    featT = feat.astype(jnp.int32).reshape(NW, BPW, SEQ).transpose(0, 2, 1)
    w_featp = jnp.pad(W_feat, ((0, 0), (0, DIM_WORD - DIM_FEAT)))
    out3 = _emb_stack(wordT, featT, W_word, w_featp)
    return jnp.transpose(out3, (2, 0, 1))


# speedup vs baseline: 11.4276x; 1.0042x over previous
import functools

import jax
import jax.numpy as jnp
from jax import lax
from jax.experimental import pallas as pl
from jax.experimental.pallas import tpu as pltpu
from jax.experimental.pallas import tpu_sc as plsc

BATCH = 4096
SEQ = 50
DIM_WORD = 128
DIM_FEAT = 64
DIM_OUT = DIM_WORD + DIM_FEAT

NUM_CORES = 2
NUM_SUBCORES = 16
NW = NUM_CORES * NUM_SUBCORES
BPW = BATCH // NW
NBUF = 3
LANES = 16
KW = DIM_WORD // LANES
KF = DIM_FEAT // LANES

_mesh = plsc.VectorSubcoreMesh(core_axis_name="c", subcore_axis_name="s")


@functools.partial(
    pl.kernel,
    mesh=_mesh,
    compiler_params=pltpu.CompilerParams(use_tc_tiling_on_sc=True,
                                         needs_layout_passes=False),
    out_type=jax.ShapeDtypeStruct((SEQ, DIM_OUT, BATCH), jnp.float32),
    scratch_types=[
        pltpu.VMEM((SEQ, BPW), jnp.int32),
        pltpu.VMEM((SEQ, BPW), jnp.int32),
        pltpu.VMEM((NBUF, BPW, DIM_WORD), jnp.float32),
        pltpu.VMEM((NBUF, BPW, DIM_WORD), jnp.float32),
        pltpu.VMEM((DIM_WORD * BPW,), jnp.float32),
        pltpu.SemaphoreType.DMA((NBUF,)),
        pltpu.SemaphoreType.DMA((NBUF,)),
        pltpu.SemaphoreType.DMA((NBUF,)),
    ],
)
def _emb_stack(word_hbm, feat_hbm, w_word_hbm, w_featp_hbm, out_hbm,
               widx, fidx, wbuf, fbuf, sw, semw, semf, sema):
    wid = lax.axis_index("s") * NUM_CORES + lax.axis_index("c")
    b0 = wid * BPW

    pltpu.sync_copy(word_hbm.at[wid], widx)
    pltpu.sync_copy(feat_hbm.at[wid], fidx)

    lane16 = lax.iota(jnp.int32, LANES)
    rowflat = [(lane16 + k * LANES) * BPW for k in range(KW)]

    def issue_gather(s, p):
        pltpu.async_copy(w_word_hbm.at[widx.at[s]], wbuf.at[p], semw.at[p])
        pltpu.async_copy(w_featp_hbm.at[fidx.at[s]], fbuf.at[p], semf.at[p])

    def wait_gather(s, p):
        pltpu.make_async_copy(w_word_hbm.at[widx.at[s]], wbuf.at[p], semw.at[p]).wait()
        pltpu.make_async_copy(w_featp_hbm.at[fidx.at[s]], fbuf.at[p], semf.at[p]).wait()

    def word_write(s, p):
        return (wbuf.at[p], out_hbm.at[s, pl.ds(0, DIM_WORD), pl.ds(b0, BPW)])

    def feat_write(s, p):
        return (fbuf.at[p, pl.ds(0, DIM_FEAT)],
                out_hbm.at[s, pl.ds(DIM_WORD, DIM_FEAT), pl.ds(b0, BPW)])

    def issue_write(s, p):
        pltpu.async_copy(*word_write(s, p), sema.at[p])
        pltpu.async_copy(*feat_write(s, p), sema.at[p])

    def wait_write(s, p):
        pltpu.make_async_copy(*word_write(s, p), sema.at[p]).wait()
        pltpu.make_async_copy(*feat_write(s, p), sema.at[p]).wait()

    def scatter_pass(buf, p, nk):
        @plsc.parallel_loop(0, BPW, step=1, unroll=2)
        def _(b):
            colx = jnp.bitwise_xor(lax.broadcast(b, (LANES,)), lane16)
            for k in range(nk):
                v = buf[p, b, pl.ds(k * LANES, LANES)]
                plsc.store_scatter(sw, [rowflat[k] + colx], v)

    def unswizzle_pass(buf, p, nd):
        @plsc.parallel_loop(0, nd, step=1, unroll=2)
        def _(d):
            dl = jnp.bitwise_and(d, LANES - 1)
            colx = jnp.bitwise_xor(lax.broadcast(dl, (LANES,)), lane16)
            row = lax.broadcast(d, (LANES,))
            for j in range(BPW // LANES):
                v = sw[pl.ds(d * BPW + j * LANES, LANES)]
                plsc.store_scatter(buf.at[p], [row, colx + j * LANES], v)

    def transpose_slab(p):
        scatter_pass(wbuf, p, KW)
        unswizzle_pass(wbuf, p, DIM_WORD)
        scatter_pass(fbuf, p, KF)
        unswizzle_pass(fbuf, p, DIM_FEAT)

    for p in range(2):
        issue_gather(p, p)

    def do_slab(s, p):
        wait_gather(s, p)
        q = (p + 2) % NBUF

        @pl.when(s >= 1)
        def _():
            issue_write(s - 1, q)

        transpose_slab(p)
        t = s + 2

        @pl.when(t < SEQ)
        def _():
            @pl.when(t >= NBUF)
            def _():
                wait_write(t - NBUF, q)
            issue_gather(t, q)

    def group_body(g, carry):
        for p in range(NBUF):
            do_slab(g * NBUF + p, p)
        return carry

    lax.fori_loop(0, SEQ // NBUF, group_body, 0)
    for p in range(SEQ - (SEQ // NBUF) * NBUF):
        do_slab((SEQ // NBUF) * NBUF + p, p)

    issue_write(SEQ - 1, (SEQ - 1) % NBUF)
    for s in range(SEQ - NBUF, SEQ):
        wait_write(s, s % NBUF)


def kernel(word, feat, W_word, W_feat):
    wordT = word.astype(jnp.int32).reshape(NW, BPW, SEQ).transpose(0, 2, 1)
    featT = feat.astype(jnp.int32).reshape(NW, BPW, SEQ).transpose(0, 2, 1)
    w_featp = jnp.pad(W_feat, ((0, 0), (0, DIM_WORD - DIM_FEAT)))
    out3 = _emb_stack(wordT, featT, W_word, w_featp)
    return jnp.transpose(out3, (2, 0, 1))
